# baseline (speedup 1.0000x reference)
import functools

import jax
import jax.numpy as jnp
from jax import lax
from jax.experimental import pallas as pl
from jax.experimental.pallas import tpu as pltpu
from jax.experimental.pallas import tpu_sc as plsc

N_NODES = 10000
D_FEAT = 128
N_EDGES = 320000

NC = 2
NS = 16
NW = NC * NS

B_W = N_EDGES // NW
CHUNK = 80
N_CHUNK = B_W // CHUNK
GFAN = 1
GROUP = CHUNK * GFAN
N_GROUP = B_W // GROUP
NBUF = 2


def _make_gather():
    mesh = plsc.VectorSubcoreMesh(
        core_axis_name="c", subcore_axis_name="s", num_cores=NC, num_subcores=NS
    )

    @functools.partial(
        pl.kernel,
        out_type=jax.ShapeDtypeStruct((N_EDGES, D_FEAT), jnp.float32),
        mesh=mesh,
        scratch_types=[
            pltpu.VMEM((N_CHUNK, CHUNK), jnp.int32),
            pltpu.VMEM((NBUF, GROUP, D_FEAT), jnp.float32),
            pltpu.VMEM_SHARED((N_NODES, D_FEAT), jnp.float32),
            pltpu.SemaphoreType.DMA((NBUF,)),
            pltpu.SemaphoreType.DMA((NBUF,)),
        ],
    )
    def gather_kernel(table_hbm, idx_hbm, out_hbm, idx_v, rows_v, table_sp,
                      gsem, ssem):
        sid = lax.axis_index("s")
        wid = sid * NC + lax.axis_index("c")
        base = wid * B_W

        stage_rows = 632
        last_rows = N_NODES - (NS - 1) * stage_rows
        for t in range(NS):
            nrows = stage_rows if t < NS - 1 else last_rows

            @pl.when(sid == t)
            def _(t=t, nrows=nrows):
                pltpu.sync_copy(
                    table_hbm.at[pl.ds(t * stage_rows, nrows)],
                    table_sp.at[pl.ds(t * stage_rows, nrows)],
                )

        pltpu.sync_copy(idx_hbm.at[wid], idx_v)
        plsc.subcore_barrier()

        def start_gathers(g, b):
            for k in range(GFAN):
                pltpu.async_copy(
                    table_sp.at[idx_v.at[g * GFAN + k]],
                    rows_v.at[b].at[pl.ds(k * CHUNK, CHUNK)],
                    gsem.at[b],
                )

        def wait_gathers(b):
            pltpu.make_async_copy(
                table_hbm.at[pl.ds(0, GROUP)], rows_v.at[b], gsem.at[b]
            ).wait()

        def store_slot(g):
            return out_hbm.at[pl.ds(base + g * GROUP, GROUP)]

        def start_store(g, b):
            pltpu.async_copy(rows_v.at[b], store_slot(g), ssem.at[b])

        def wait_store(g, b):
            pltpu.make_async_copy(rows_v.at[b], store_slot(g), ssem.at[b]).wait()

        for b in range(NBUF):
            start_gathers(b, b)

        @pl.loop(0, N_GROUP - 1, step=NBUF)
        def _ring(g0):
            for b in range(NBUF):
                g = g0 + b
                wait_gathers(b)
                start_store(g, b)
                wait_store(g, b)

                @pl.when(g + NBUF < N_GROUP)
                def _():
                    start_gathers(g + NBUF, b)

        g_last = N_GROUP - 1
        b_last = g_last % NBUF
        wait_gathers(b_last)
        start_store(g_last, b_last)
        wait_store(g_last, b_last)

    return gather_kernel


_gather = _make_gather()


@jax.jit
def kernel(V_set, node_ids):
    table = V_set[0]
    idx = node_ids.reshape(NW, N_CHUNK, CHUNK)
    out = _gather(table, idx)
    return out[jnp.newaxis]


# speedup vs baseline: 1.2003x; 1.2003x over previous
import functools

import jax
import jax.numpy as jnp
from jax import lax
from jax.experimental import pallas as pl
from jax.experimental.pallas import tpu as pltpu
from jax.experimental.pallas import tpu_sc as plsc

N_NODES = 10000
D_FEAT = 128
N_EDGES = 320000

NC = 2
NS = 16
NW = NC * NS

B_W = N_EDGES // NW
CHUNK = 80
N_CHUNK = B_W // CHUNK
GFAN = 1
GROUP = CHUNK * GFAN
N_GROUP = B_W // GROUP
NBUF = 3


def _make_gather():
    mesh = plsc.VectorSubcoreMesh(
        core_axis_name="c", subcore_axis_name="s", num_cores=NC, num_subcores=NS
    )

    @functools.partial(
        pl.kernel,
        out_type=jax.ShapeDtypeStruct((N_EDGES, D_FEAT), jnp.float32),
        mesh=mesh,
        scratch_types=[
            pltpu.VMEM((N_CHUNK, CHUNK), jnp.int32),
            pltpu.VMEM((NBUF, GROUP, D_FEAT), jnp.float32),
            pltpu.VMEM_SHARED((N_NODES, D_FEAT), jnp.float32),
            pltpu.SemaphoreType.DMA((NBUF,)),
            pltpu.SemaphoreType.DMA((NBUF,)),
        ],
    )
    def gather_kernel(table_hbm, idx_hbm, out_hbm, idx_v, rows_v, table_sp,
                      gsem, ssem):
        sid = lax.axis_index("s")
        wid = sid * NC + lax.axis_index("c")
        base = wid * B_W

        @pl.when(sid == NS - 1)
        def _():
            pltpu.sync_copy(table_hbm, table_sp)

        pltpu.sync_copy(idx_hbm.at[wid], idx_v)
        plsc.subcore_barrier()

        def start_gathers(g, b):
            for k in range(GFAN):
                pltpu.async_copy(
                    table_sp.at[idx_v.at[g * GFAN + k]],
                    rows_v.at[b].at[pl.ds(k * CHUNK, CHUNK)],
                    gsem.at[b],
                )

        def wait_gathers(b):
            pltpu.make_async_copy(
                table_hbm.at[pl.ds(0, GROUP)], rows_v.at[b], gsem.at[b]
            ).wait()

        def store_slot(g):
            return out_hbm.at[pl.ds(base + g * GROUP, GROUP)]

        def start_store(g, b):
            pltpu.async_copy(rows_v.at[b], store_slot(g), ssem.at[b])

        def wait_store(g, b):
            pltpu.make_async_copy(rows_v.at[b], store_slot(g), ssem.at[b]).wait()

        for b in range(NBUF - 1):
            start_gathers(b, b)

        @pl.loop(0, N_GROUP - (NBUF - 1), step=NBUF)
        def _ring(g0):
            for b in range(NBUF):
                g = g0 + b
                wait_gathers(b)
                start_store(g, b)

                @pl.when(g > 0)
                def _():
                    wait_store(g - 1, (g - 1) % NBUF)

                @pl.when(g + NBUF - 1 < N_GROUP)
                def _():
                    start_gathers(g + NBUF - 1, (g + NBUF - 1) % NBUF)

        for g in range(N_GROUP - (NBUF - 1), N_GROUP):
            b = g % NBUF
            wait_gathers(b)
            start_store(g, b)
            wait_store(g - 1, (g - 1) % NBUF)

        wait_store(N_GROUP - 1, (N_GROUP - 1) % NBUF)

    return gather_kernel


_gather = _make_gather()


@jax.jit
def kernel(V_set, node_ids):
    table = V_set[0]
    idx = node_ids.reshape(NW, N_CHUNK, CHUNK)
    out = _gather(table, idx)
    return out[jnp.newaxis]
